# baseline (speedup 1.0000x reference)
import functools

import jax
import jax.numpy as jnp
from jax import lax
from jax.experimental import pallas as pl
from jax.experimental.pallas import tpu as pltpu
from jax.experimental.pallas import tpu_sc as plsc

NC = 1
NS = 16
L = 16
CH = 2


def _make_gather(B: int, N: int, D: int):
  NW = NC * NS
  assert B % (8 * NW * CH) == 0 and D % L == 0
  b_per_w = B // NW
  b_per_c = b_per_w // CH
  mesh = plsc.VectorSubcoreMesh(
      core_axis_name="c", subcore_axis_name="s", num_cores=NC, num_subcores=NS
  )

  @functools.partial(
      pl.kernel,
      mesh=mesh,
      out_type=jax.ShapeDtypeStruct((B, D), jnp.float32),
      scratch_types=[
          pltpu.VMEM((b_per_w,), jnp.int32),
          pltpu.VMEM((CH, b_per_c, D), jnp.float32),
          pltpu.SemaphoreType.DMA,
          pltpu.SemaphoreType.DMA,
      ],
  )
  def gather(table_hbm, idx_hbm, out_hbm, idx_v, rows_v, sem0, sem1):
    wid = lax.axis_index("s") * NC + lax.axis_index("c")
    base = wid * b_per_w
    pltpu.sync_copy(idx_hbm.at[pl.ds(base, b_per_w)], idx_v)
    for j in range(b_per_w // L):
      sl = pl.ds(j * L, L)
      b_ids = lax.iota(jnp.int32, L) + (base + j * L)
      idx_v[sl] = idx_v[sl] + b_ids * N
    sems = (sem0, sem1)
    gathers = []
    for c in range(CH):
      g = pltpu.async_copy(
          table_hbm.at[idx_v.at[pl.ds(c * b_per_c, b_per_c)]],
          rows_v.at[c], sems[c])
      gathers.append(g)
    for c in range(CH):
      gathers[c].wait()
      pltpu.sync_copy(rows_v.at[c], out_hbm.at[pl.ds(base + c * b_per_c, b_per_c)])

  return gather


def kernel(embeddings, current_node):
  B, N, D = embeddings.shape
  table = embeddings.reshape(B * N, D)
  idx = current_node.astype(jnp.int32)
  return _make_gather(B, N, D)(table, idx)
